# baseline (speedup 1.0000x reference)
import functools

import jax
import jax.numpy as jnp
from jax import lax
from jax.experimental import pallas as pl
from jax.experimental.pallas import tpu as pltpu
from jax.experimental.pallas import tpu_sc as plsc

_NC = 2
_NS = 16
_L = 16
_NW = _NC * _NS
_NBUF = 4
_LA = 2


@functools.lru_cache(maxsize=None)
def _make_pool(S, B, V2, W):
    NB = B // _NW
    mesh = plsc.VectorSubcoreMesh(core_axis_name="c", subcore_axis_name="s")

    def body(x_hbm, table_hbm, out_hbm, idx_v, idxu_r, idxt_r, rows_v,
             acc_sh, sem_g, sem_s):
        cid = lax.axis_index("c")
        sid = lax.axis_index("s")
        wid = sid * _NC + cid
        base = wid * NB
        abase = sid * (2 * NB)

        pltpu.sync_copy(x_hbm.at[:, pl.ds(base, NB)], idx_v)

        def fill_rings(s, k):
            for j in range(NB // _L):
                sl = pl.ds(j * _L, _L)
                v = idx_v[s, sl]
                lo = v < V2
                r2 = abase + 2 * (j * _L + lax.iota(jnp.int32, _L))
                idxu_r[k, sl] = jnp.where(lo, v, v - V2)
                idxt_r[k, sl] = jnp.where(lo, r2, r2 + 1)

        zero = jnp.zeros((_L,), jnp.float32)

        def zrow(r, carry):
            for j in range(W // _L):
                rows_v[0, r, pl.ds(j * _L, _L)] = zero
            return carry

        lax.fori_loop(0, NB, zrow, 0)
        pltpu.sync_copy(rows_v.at[0], acc_sh.at[pl.ds(abase, NB)])
        pltpu.sync_copy(rows_v.at[0], acc_sh.at[pl.ds(abase + NB, NB)])

        def issue_gather(b):
            pltpu.async_copy(table_hbm.at[idxu_r.at[b]], rows_v.at[b], sem_g)

        def wait_gather(b):
            pltpu.make_async_copy(
                table_hbm.at[idxu_r.at[0]], rows_v.at[b], sem_g
            ).wait()

        def issue_scatter(b):
            pltpu.async_copy(rows_v.at[b], acc_sh.at[idxt_r.at[b]], sem_s, add=True)

        def drain_scatter():
            pltpu.make_async_copy(rows_v.at[0], acc_sh.at[idxt_r.at[0]], sem_s).wait()

        for s in range(_LA):
            fill_rings(s, s % _NBUF)
            issue_gather(s % _NBUF)

        for s in range(2):
            wait_gather(s % _NBUF)
            issue_scatter(s % _NBUF)
            fill_rings(s + _LA, (s + _LA) % _NBUF)
            issue_gather((s + _LA) % _NBUF)

        def step(s, b, b2):
            wait_gather(b)
            issue_scatter(b)
            drain_scatter()
            fill_rings(s + _LA, b2)
            issue_gather(b2)

        def loop_body(g, carry):
            s0 = _NBUF * g
            for j in range(_NBUF):
                step(s0 + j, j, (j + _LA) % _NBUF)
            return carry

        GHI = (S - _LA) // _NBUF
        for s in range(2, _NBUF):
            step(s, s % _NBUF, (s + _LA) % _NBUF)
        lax.fori_loop(1, GHI, loop_body, 0)
        for s in range(_NBUF * GHI, S):
            b = s % _NBUF
            wait_gather(b)
            issue_scatter(b)
            drain_scatter()
            if s + _LA < S:
                fill_rings(s + _LA, (s + _LA) % _NBUF)
                issue_gather((s + _LA) % _NBUF)
        drain_scatter()
        drain_scatter()

        pltpu.sync_copy(acc_sh.at[pl.ds(abase, NB)], rows_v.at[0])
        pltpu.sync_copy(acc_sh.at[pl.ds(abase + NB, NB)], rows_v.at[1])
        E = W // 2

        def combine(r, carry):
            for h in range(2):
                for j in range(E // _L):
                    sl = pl.ds(j * _L, _L)
                    left = rows_v[h, 2 * r + 0, sl]
                    right = rows_v[h, 2 * r + 1, pl.ds(E + j * _L, _L)]
                    rows_v[2 + h, (NB // 2) * h + r, sl] = left + right
            return carry

        lax.fori_loop(0, NB // 2, combine, 0)
        pltpu.sync_copy(rows_v.at[2, pl.ds(0, NB // 2)],
                        out_hbm.at[pl.ds(base, NB // 2)])
        pltpu.sync_copy(rows_v.at[3, pl.ds(NB // 2, NB // 2)],
                        out_hbm.at[pl.ds(base + NB // 2, NB // 2)])

    return pl.kernel(
        body,
        out_type=jax.ShapeDtypeStruct((B, W), jnp.float32),
        mesh=mesh,
        scratch_types=[
            pltpu.VMEM((S, NB), jnp.int32),
            pltpu.VMEM((_NBUF, NB), jnp.int32),
            pltpu.VMEM((_NBUF, NB), jnp.int32),
            pltpu.VMEM((_NBUF, NB, W), jnp.float32),
            pltpu.VMEM_SHARED((_NS * 2 * NB, W), jnp.float32),
            pltpu.SemaphoreType.DMA,
            pltpu.SemaphoreType.DMA,
        ],
        compiler_params=pltpu.CompilerParams(use_tc_tiling_on_sc=True),
    )


@functools.lru_cache(maxsize=None)
def _make_pack(V2, E):
    R = 5000
    NBLK = V2 // R

    def body(a_ref, o_ref):
        h = pl.program_id(1)

        @pl.when(h == 0)
        def _():
            o_ref[:, :E] = a_ref[...]

        @pl.when(h == 1)
        def _():
            o_ref[:, E:] = a_ref[...]

    return pl.pallas_call(
        body,
        grid=(NBLK, 2),
        in_specs=[pl.BlockSpec((R, E), lambda i, h: (i + h * NBLK, 0))],
        out_specs=pl.BlockSpec((R, 2 * E), lambda i, h: (i, 0)),
        out_shape=jax.ShapeDtypeStruct((V2, 2 * E), jnp.float32),
    )


@functools.lru_cache(maxsize=None)
def _make_mlp(B, E, H, C):
    def body(p_ref, w1_ref, b1_ref, w2_ref, b2_ref, o_ref):
        h = jnp.dot(p_ref[:, :E], w1_ref[...], preferred_element_type=jnp.float32)
        h = jnp.maximum(h + b1_ref[...], 0.0)
        logits = jnp.dot(h, w2_ref[...], preferred_element_type=jnp.float32)
        logits = logits + b2_ref[...]
        m = jnp.max(logits, axis=1, keepdims=True)
        e = jnp.exp(logits - m)
        o_ref[...] = e / jnp.sum(e, axis=1, keepdims=True)

    return pl.pallas_call(
        body,
        out_shape=jax.ShapeDtypeStruct((B, C), jnp.float32),
    )


def kernel(x, table, W1, b1, W2, b2):
    S, B = x.shape
    V, E = table.shape
    H = W1.shape[1]
    C = W2.shape[1]
    t2 = _make_pack(V // 2, E)(table)
    pooled = _make_pool(S, B, V // 2, 2 * E)(x, t2)
    return _make_mlp(B, E, H, C)(
        pooled, W1 * (1.0 / S), b1.reshape(1, H), W2, b2.reshape(1, C)
    )


# speedup vs baseline: 1.6117x; 1.6117x over previous
import functools

import jax
import jax.numpy as jnp
from jax import lax
from jax.experimental import pallas as pl
from jax.experimental.pallas import tpu as pltpu
from jax.experimental.pallas import tpu_sc as plsc

_NC = 2
_NS = 16
_L = 16
_NW = _NC * _NS
_NBUF = 6
_LA = 4


@functools.lru_cache(maxsize=None)
def _make_pack(V, E):
    C = 2048
    NBLK = -(-V // C)

    def body(a_ref, o_ref):
        a = a_ref[...]
        o_ref[:, :E] = jnp.swapaxes(a[:, : C // 2], 0, 1)
        o_ref[:, E:] = jnp.swapaxes(a[:, C // 2:], 0, 1)

    return pl.pallas_call(
        body,
        grid=(NBLK,),
        in_specs=[pl.BlockSpec((E, C), lambda i: (0, i))],
        out_specs=pl.BlockSpec((C // 2, 2 * E), lambda i: (i, 0)),
        out_shape=jax.ShapeDtypeStruct((NBLK * (C // 2), 2 * E), jnp.float32),
    )


@functools.lru_cache(maxsize=None)
def _make_pool(S, B, V, E):
    NB = B // _NW
    mesh = plsc.VectorSubcoreMesh(core_axis_name="c", subcore_axis_name="s")

    def body(x_hbm, table_hbm, out_hbm, idx_v, idxb_v, rows_v, acc_sh, sem_g, sem_s):
        cid = lax.axis_index("c")
        sid = lax.axis_index("s")
        wid = sid * _NC + cid
        base = wid * NB
        sbase = sid * NB

        pltpu.sync_copy(x_hbm.at[:, pl.ds(base, NB)], idx_v)

        def remap(s, carry):
            for j in range(NB // _L):
                sl = pl.ds(j * _L, _L)
                v = idx_v[s, sl]
                u = (
                    lax.shift_left(lax.shift_right_logical(v, 11), 11)
                    + lax.shift_left(v & 1023, 1)
                    + (lax.shift_right_logical(v, 10) & 1)
                )
                idx_v[s, sl] = u
            return carry

        lax.fori_loop(0, S, remap, 0)

        for j in range(NB // _L):
            idxb_v[pl.ds(j * _L, _L)] = sbase + j * _L + lax.iota(jnp.int32, _L)

        zero = jnp.zeros((_L,), jnp.float32)

        def zrow(r, carry):
            for j in range(E // _L):
                rows_v[0, r, pl.ds(j * _L, _L)] = zero
            return carry

        lax.fori_loop(0, NB, zrow, 0)
        pltpu.sync_copy(rows_v.at[0], acc_sh.at[pl.ds(sbase, NB)])

        def issue_gather(s, b):
            pltpu.async_copy(table_hbm.at[idx_v.at[s]], rows_v.at[b], sem_g)

        def wait_gather(b):
            pltpu.make_async_copy(
                table_hbm.at[idx_v.at[0]], rows_v.at[b], sem_g
            ).wait()

        def issue_scatter(b):
            pltpu.async_copy(rows_v.at[b], acc_sh.at[idxb_v], sem_s, add=True)

        def drain_scatter():
            pltpu.make_async_copy(rows_v.at[0], acc_sh.at[idxb_v], sem_s).wait()

        for s in range(_LA):
            issue_gather(s, s % _NBUF)

        for s in range(_NBUF):
            wait_gather(s % _NBUF)
            issue_scatter(s % _NBUF)
            if s >= 2:
                drain_scatter()
            issue_gather(s + _LA, (s + _LA) % _NBUF)

        def loop_body(g, carry):
            s0 = _NBUF * g
            for j in range(_NBUF):
                wait_gather(j)
                issue_scatter(j)
                drain_scatter()
                issue_gather(s0 + j + _LA, (j + _LA) % _NBUF)
            return carry

        GHI = (S - 2 * _NBUF) // _NBUF + 1
        lax.fori_loop(1, GHI, loop_body, 0)

        for s in range(_NBUF * GHI, S):
            b = s % _NBUF
            wait_gather(b)
            issue_scatter(b)
            drain_scatter()
            if s + _LA < S:
                issue_gather(s + _LA, (s + _LA) % _NBUF)
        drain_scatter()
        drain_scatter()

        pltpu.sync_copy(acc_sh.at[pl.ds(sbase, NB)], out_hbm.at[pl.ds(base, NB)])

    return pl.kernel(
        body,
        out_type=jax.ShapeDtypeStruct((B, E), jnp.float32),
        mesh=mesh,
        scratch_types=[
            pltpu.VMEM((S, NB), jnp.int32),
            pltpu.VMEM((NB,), jnp.int32),
            pltpu.VMEM((_NBUF, NB, E), jnp.float32),
            pltpu.VMEM_SHARED((_NS * NB, E), jnp.float32),
            pltpu.SemaphoreType.DMA,
            pltpu.SemaphoreType.DMA,
        ],
        compiler_params=pltpu.CompilerParams(use_tc_tiling_on_sc=False),
    )


@functools.lru_cache(maxsize=None)
def _make_mlp(B, E, H, C):
    def body(p_ref, w1_ref, b1_ref, w2_ref, b2_ref, o_ref):
        h = jnp.dot(p_ref[...], w1_ref[...], preferred_element_type=jnp.float32)
        h = jnp.maximum(h + b1_ref[...], 0.0)
        logits = jnp.dot(h, w2_ref[...], preferred_element_type=jnp.float32)
        logits = logits + b2_ref[...]
        m = jnp.max(logits, axis=1, keepdims=True)
        e = jnp.exp(logits - m)
        o_ref[...] = e / jnp.sum(e, axis=1, keepdims=True)

    return pl.pallas_call(
        body,
        out_shape=jax.ShapeDtypeStruct((B, C), jnp.float32),
    )


def kernel(x, table, W1, b1, W2, b2):
    S, B = x.shape
    V, E = table.shape
    H = W1.shape[1]
    C = W2.shape[1]
    tT = jnp.swapaxes(table, 0, 1)
    t2 = _make_pack(V, E)(tT)
    VP = t2.shape[0] * 2
    t3 = t2.reshape(VP, E)
    pooled = _make_pool(S, B, VP, E)(x, t3)
    return _make_mlp(B, E, H, C)(
        pooled, W1 * (1.0 / S), b1.reshape(1, H), W2, b2.reshape(1, C)
    )
